# baseline (device time: 120727 ns/iter reference)
import jax
import jax.numpy as jnp
from jax import lax
from jax.experimental import pallas as pl
from jax.experimental.pallas import tpu as pltpu

T = 4096
N = 1024
U = 4


def kernel(x, dest):
    my_x = lax.axis_index("x")
    dest = dest.astype(jnp.int32)

    iota = jnp.arange(T, dtype=jnp.int32)
    ones_before = (jnp.cumsum(dest) - dest).astype(jnp.int32)
    zeros_before = iota - ones_before
    c0 = (T - ones_before[-1] - dest[-1]).astype(jnp.int32)
    slot = jnp.where(dest == 0, zeros_before, c0 + ones_before)
    order = (
        jnp.zeros(T, jnp.int32)
        .at[slot]
        .set(iota, unique_indices=True, indices_are_sorted=False)
    )

    is0 = my_x == 0
    L = jnp.where(is0, T - c0, c0).astype(jnp.int32)
    keep = (T - L).astype(jnp.int32)
    base_keep = jnp.where(is0, 0, c0).astype(jnp.int32)
    base_send = jnp.where(is0, 0, T - c0).astype(jnp.int32)
    ko = jnp.where(is0, 0, c0).astype(jnp.int32)
    so = jnp.where(is0, c0, 0).astype(jnp.int32)

    def chunks(n):
        return [n // 512, (n % 512) // 8, n % 8]

    meta = jnp.stack(
        [L, keep, base_keep, base_send, ko, so, *chunks(L), *chunks(keep)]
    )
    x4 = x.reshape(T // 8, 8, 8, N // 8).transpose(0, 2, 1, 3)

    def body(meta_ref, order_ref, x_ref, out_ref, send_sem, recv_sem, copy_sem):
        nbr = (1 - lax.axis_index("x"), lax.axis_index("y"))

        barrier = pltpu.get_barrier_semaphore()
        pl.semaphore_signal(
            barrier, inc=1, device_id=nbr, device_id_type=pl.DeviceIdType.MESH
        )
        pl.semaphore_wait(barrier, 1)

        L_ = meta_ref[0]
        keep_ = meta_ref[1]
        base_keep_ = meta_ref[2]
        base_send_ = meta_ref[3]
        ko_ = meta_ref[4]
        so_ = meta_ref[5]

        def row_slice(ref, j):
            return ref.at[pl.ds(j // 8, 1), :, pl.ds(j % 8, 1), :]

        def send_one(i):
            src = order_ref[so_ + i]
            pltpu.make_async_remote_copy(
                src_ref=row_slice(x_ref, src),
                dst_ref=row_slice(out_ref, base_send_ + i),
                send_sem=send_sem,
                recv_sem=recv_sem,
                device_id=nbr,
                device_id_type=pl.DeviceIdType.MESH,
            ).start()

        def keep_one(i):
            src = order_ref[ko_ + i]
            pltpu.make_async_copy(
                row_slice(x_ref, src),
                row_slice(out_ref, base_keep_ + i),
                copy_sem,
            ).start()

        def unrolled(issue, count):
            def block(t, _):
                for u in range(U):
                    issue(t * U + u)
                return 0

            def tail(i, _):
                issue(i)
                return 0

            lax.fori_loop(0, count // U, block, 0)
            lax.fori_loop((count // U) * U, count, tail, 0)

        unrolled(send_one, L_)
        unrolled(keep_one, keep_)

        def drain(sem_wait, counts_at):
            for sz, k in zip((512, 8, 1), counts_at):
                def w(_, __, sz=sz):
                    sem_wait((pl.ds(0, sz), slice(None), pl.ds(0, 1)))
                    return 0
                lax.fori_loop(0, meta_ref[k], w, 0)

        def wait_copy(sl):
            pltpu.make_async_copy(
                x_ref.at[sl], out_ref.at[sl], copy_sem
            ).wait()

        def wait_remote(kind):
            def w(sl):
                d = pltpu.make_async_remote_copy(
                    src_ref=x_ref.at[sl],
                    dst_ref=out_ref.at[sl],
                    send_sem=send_sem,
                    recv_sem=recv_sem,
                    device_id=nbr,
                    device_id_type=pl.DeviceIdType.MESH,
                )
                if kind == "recv":
                    d.wait_recv()
                else:
                    d.wait_send()
            return w

        drain(wait_copy, (9, 10, 11))
        drain(wait_remote("recv"), (6, 7, 8))
        drain(wait_remote("send"), (6, 7, 8))

    out = pl.pallas_call(
        body,
        out_shape=jax.ShapeDtypeStruct((T // 8, 8, 8, N // 8), jnp.float32),
        in_specs=[
            pl.BlockSpec(memory_space=pltpu.SMEM),
            pl.BlockSpec(memory_space=pltpu.SMEM),
            pl.BlockSpec(memory_space=pltpu.HBM),
        ],
        out_specs=pl.BlockSpec(memory_space=pltpu.HBM),
        scratch_shapes=[
            pltpu.SemaphoreType.DMA,
            pltpu.SemaphoreType.DMA,
            pltpu.SemaphoreType.DMA,
        ],
        compiler_params=pltpu.CompilerParams(collective_id=0),
    )(meta, order, x4)
    return out.transpose(0, 2, 1, 3).reshape(T, N)


# device time: 107349 ns/iter; 1.1246x vs baseline; 1.1246x over previous
import jax
import jax.numpy as jnp
from jax import lax
from jax.experimental import pallas as pl
from jax.experimental.pallas import tpu as pltpu

T = 4096
N = 1024
U = 8


def kernel(x, dest):
    my_x = lax.axis_index("x")
    dest = dest.astype(jnp.int32)

    packed = dest * T + jnp.arange(T, dtype=jnp.int32)
    order = (jnp.sort(packed) & (T - 1)).astype(jnp.int32)
    c0 = jnp.sum(dest == 0).astype(jnp.int32)

    is0 = my_x == 0
    L = jnp.where(is0, T - c0, c0).astype(jnp.int32)
    keep = (T - L).astype(jnp.int32)
    base_keep = jnp.where(is0, 0, c0).astype(jnp.int32)
    base_send = jnp.where(is0, 0, T - c0).astype(jnp.int32)
    ko = jnp.where(is0, 0, c0).astype(jnp.int32)
    so = jnp.where(is0, c0, 0).astype(jnp.int32)

    def chunks(n):
        return [n // 512, (n % 512) // 8, n % 8]

    meta = jnp.stack(
        [L, keep, base_keep, base_send, ko, so, *chunks(L), *chunks(keep)]
    )
    x4 = x.reshape(T // 8, 8, 8, N // 8).transpose(0, 2, 1, 3)

    def body(meta_ref, order_ref, x_ref, out_ref, send_sem, recv_sem, copy_sem):
        nbr = (1 - lax.axis_index("x"), lax.axis_index("y"))

        barrier = pltpu.get_barrier_semaphore()
        pl.semaphore_signal(
            barrier, inc=1, device_id=nbr, device_id_type=pl.DeviceIdType.MESH
        )
        pl.semaphore_wait(barrier, 1)

        L_ = meta_ref[0]
        keep_ = meta_ref[1]
        base_keep_ = meta_ref[2]
        base_send_ = meta_ref[3]
        ko_ = meta_ref[4]
        so_ = meta_ref[5]

        def row_slice(ref, j):
            return ref.at[pl.ds(j // 8, 1), :, pl.ds(j % 8, 1), :]

        def send_one(i):
            src = order_ref[so_ + i]
            pltpu.make_async_remote_copy(
                src_ref=row_slice(x_ref, src),
                dst_ref=row_slice(out_ref, base_send_ + i),
                send_sem=send_sem,
                recv_sem=recv_sem,
                device_id=nbr,
                device_id_type=pl.DeviceIdType.MESH,
            ).start()

        def keep_one(i):
            src = order_ref[ko_ + i]
            pltpu.make_async_copy(
                row_slice(x_ref, src),
                row_slice(out_ref, base_keep_ + i),
                copy_sem,
            ).start()

        def unrolled(issue, count):
            def block(t, _):
                for u in range(U):
                    issue(t * U + u)
                return 0

            def tail(i, _):
                issue(i)
                return 0

            lax.fori_loop(0, count // U, block, 0)
            lax.fori_loop((count // U) * U, count, tail, 0)

        unrolled(send_one, L_)
        unrolled(keep_one, keep_)

        def drain(sem_wait, counts_at):
            for sz, k in zip((512, 8, 1), counts_at):
                def w(_, __, sz=sz):
                    sem_wait((pl.ds(0, sz), slice(None), pl.ds(0, 1)))
                    return 0
                lax.fori_loop(0, meta_ref[k], w, 0)

        def wait_copy(sl):
            pltpu.make_async_copy(
                x_ref.at[sl], out_ref.at[sl], copy_sem
            ).wait()

        def wait_remote(kind):
            def w(sl):
                d = pltpu.make_async_remote_copy(
                    src_ref=x_ref.at[sl],
                    dst_ref=out_ref.at[sl],
                    send_sem=send_sem,
                    recv_sem=recv_sem,
                    device_id=nbr,
                    device_id_type=pl.DeviceIdType.MESH,
                )
                if kind == "recv":
                    d.wait_recv()
                else:
                    d.wait_send()
            return w

        drain(wait_copy, (9, 10, 11))
        drain(wait_remote("recv"), (6, 7, 8))
        drain(wait_remote("send"), (6, 7, 8))

    out = pl.pallas_call(
        body,
        out_shape=jax.ShapeDtypeStruct((T // 8, 8, 8, N // 8), jnp.float32),
        in_specs=[
            pl.BlockSpec(memory_space=pltpu.SMEM),
            pl.BlockSpec(memory_space=pltpu.SMEM),
            pl.BlockSpec(memory_space=pltpu.HBM),
        ],
        out_specs=pl.BlockSpec(memory_space=pltpu.HBM),
        scratch_shapes=[
            pltpu.SemaphoreType.DMA,
            pltpu.SemaphoreType.DMA,
            pltpu.SemaphoreType.DMA,
        ],
        compiler_params=pltpu.CompilerParams(collective_id=0),
    )(meta, order, x4)
    return out.transpose(0, 2, 1, 3).reshape(T, N)
